# baseline (speedup 1.0000x reference)
import functools

import jax
import jax.numpy as jnp
from jax import lax
from jax.experimental import pallas as pl
from jax.experimental.pallas import tpu as pltpu
from jax.experimental.pallas import tpu_sc as plsc

GROUP = 5
RCH = 8
SLAB = 8
CPS = SLAB * GROUP // RCH
PAIR_CH = 2 * CPS
LANES = 16

_SEGS = []
for _c in range(CPS):
    segs = []
    r0, r1 = RCH * _c, RCH * _c + RCH
    g0, g1 = r0 // GROUP, (r1 - 1) // GROUP
    for _g in range(g0, g1 + 1):
        lo = max(r0, _g * GROUP)
        hi = min(r1, _g * GROUP + GROUP)
        segs.append((_g, lo - r0, hi - r0, lo == _g * GROUP))
    _SEGS.append(segs)


def _sc_gather_pool(table, idx_flat, ng, d, nw):
    mesh = plsc.VectorSubcoreMesh(core_axis_name="c", subcore_axis_name="s")
    n_pairs = ng // (2 * SLAB)
    base = n_pairs // nw
    extra = n_pairs - base * nw
    max_pairs = base + (1 if extra else 0)
    win = max_pairs * PAIR_CH * RCH
    nvec = d // LANES

    @functools.partial(
        pl.kernel,
        mesh=mesh,
        out_type=jax.ShapeDtypeStruct((ng * d,), jnp.float32),
        scratch_types=[
            pltpu.VMEM((win,), jnp.int32),
            pltpu.VMEM((RCH, d), jnp.float32),
            pltpu.VMEM((RCH, d), jnp.float32),
            pltpu.VMEM((SLAB * d,), jnp.float32),
            pltpu.VMEM((SLAB * d,), jnp.float32),
            pltpu.SemaphoreType.DMA,
            pltpu.SemaphoreType.DMA,
            pltpu.SemaphoreType.DMA,
            pltpu.SemaphoreType.DMA,
        ],
    )
    def pool_kernel(
        table_hbm, idx_hbm, out_hbm,
        idx_v, rows0, rows1, slab0, slab1, sem0, sem1, ssem0, ssem1,
    ):
        rows = (rows0, rows1)
        sems = (sem0, sem1)
        slabs = (slab0, slab1)
        ssems = (ssem0, ssem1)
        wid = lax.axis_index("s") * 2 + lax.axis_index("c")
        start = base * wid + jnp.minimum(wid, extra)
        my_pairs = base + jnp.where(wid < extra, 1, 0)
        total_chunks = my_pairs * PAIR_CH
        pltpu.sync_copy(idx_hbm.at[pl.ds(start * (PAIR_CH * RCH), win)], idx_v)

        def fire(c, b):
            pltpu.async_copy(
                table_hbm.at[idx_v.at[pl.ds(c * RCH, RCH)]], rows[b], sems[b]
            )

        def wait(c, b):
            pltpu.make_async_copy(
                table_hbm.at[idx_v.at[pl.ds(c * RCH, RCH)]], rows[b], sems[b]
            ).wait()

        def flush(slab_id, sb):
            pltpu.async_copy(
                slabs[sb],
                out_hbm.at[pl.ds(slab_id * (SLAB * d), SLAB * d)],
                ssems[sb],
            )

        def flush_wait(slab_id, sb):
            pltpu.make_async_copy(
                slabs[sb],
                out_hbm.at[pl.ds(slab_id * (SLAB * d), SLAB * d)],
                ssems[sb],
            ).wait()

        fire(0, 0)

        def do_pair(p, carry):
            for sp in range(2):
                sb = sp
                @pl.when(p > 0)
                def _(sb=sb):
                    flush_wait((start + p - 1) * 2 + sp, sb)

                for c in range(CPS):
                    j = sp * CPS + c
                    cg = p * PAIR_CH + j
                    b = j % 2
                    nxt = cg + 1

                    @pl.when(nxt < total_chunks)
                    def _(nxt=nxt, b=b):
                        fire(nxt, 1 - b)

                    wait(cg, b)
                    rv = rows[b]
                    sv = slabs[sb]
                    segs = _SEGS[c]

                    def vbody(i, _, rv=rv, sv=sv, segs=segs):
                        for u in range(2):
                            o = (2 * i + u) * LANES
                            for slot, lo, hi, first in segs:
                                acc = rv[lo, pl.ds(o, LANES)]
                                for r in range(lo + 1, hi):
                                    acc = acc + rv[r, pl.ds(o, LANES)]
                                acc = acc * (1.0 / GROUP)
                                if first:
                                    sv[pl.ds(slot * d + o, LANES)] = acc
                                else:
                                    sv[pl.ds(slot * d + o, LANES)] = (
                                        sv[pl.ds(slot * d + o, LANES)] + acc
                                    )
                        return _

                    lax.fori_loop(0, nvec // 2, vbody, 0)

                flush((start + p) * 2 + sp, sb)
            return carry

        lax.fori_loop(0, my_pairs, do_pair, 0)

        for sp in range(2):
            flush_wait((start + my_pairs - 1) * 2 + sp, sp)

    return pool_kernel(table, idx_flat)


def _tc_matmul(pooled, w_bf16, ng, d, bm, bk):
    def body(a_ref, w_ref, o_ref):
        k = pl.program_id(1)
        a = a_ref[...].astype(jnp.bfloat16)
        part = lax.dot_general(
            a,
            w_ref[...],
            (((1,), (1,)), ((), ())),
            preferred_element_type=jnp.float32,
        )

        @pl.when(k == 0)
        def _():
            o_ref[...] = part

        @pl.when(k != 0)
        def _():
            o_ref[...] += part

    return pl.pallas_call(
        body,
        grid=(ng // bm, d // bk),
        in_specs=[
            pl.BlockSpec((bm, bk), lambda i, k: (i, k)),
            pl.BlockSpec((d, bk), lambda i, k: (0, k)),
        ],
        out_specs=pl.BlockSpec((bm, d), lambda i, k: (i, 0)),
        out_shape=jax.ShapeDtypeStruct((ng, d), jnp.float32),
    )(pooled, w_bf16)


def kernel(audio_ids, embed_table, W_out):
    b, s = audio_ids.shape
    v, d = embed_table.shape
    ng = (b * s) // GROUP
    nw = 32

    ids = audio_ids.reshape(-1).astype(jnp.int32)
    n_pairs = ng // (2 * SLAB)
    max_pairs = n_pairs // nw + (1 if n_pairs % nw else 0)
    pad = nw * max_pairs * PAIR_CH * RCH - ids.shape[0]
    idx_flat = jnp.concatenate([ids, jnp.zeros((pad,), jnp.int32)])

    pooled = _sc_gather_pool(embed_table, idx_flat, ng, d, nw).reshape(ng, d)
    out = _tc_matmul(pooled, W_out.astype(jnp.bfloat16), ng, d, bm=800, bk=512)
    return out.reshape(b, s // GROUP, d)
